# Initial kernel scaffold; baseline (speedup 1.0000x reference)
import jax
import jax.numpy as jnp
from jax.experimental import pallas as pl


def kernel(x, pos_edge_index, edge_attr, target_node_embeddings, params):
    raise NotImplementedError("write your pallas kernel here")


# speedup vs baseline: 266.6844x; 266.6844x over previous
import functools

import jax
import jax.numpy as jnp
from jax import lax
from jax.experimental import pallas as pl
from jax.experimental.pallas import tpu as pltpu
from jax.experimental.pallas import tpu_sc as plsc

H = 4
C = 32
D = H * C
NS = 1024
NT = 256
E = 32768
INV = 1.0 / (C ** 0.5)
HEADS_BASE = NS * NS
EPT = E // 8
ROWS = EPT // 128


def _k1_body(x_ref, w4_ref, b4_ref, we_ref, ei_ref,
             qkf_ref, m_ref, u_ref, v4_ref, skipt_ref, idx_ref):
    x = x_ref[...]
    w4 = w4_ref[...]
    qkvst = lax.dot_general(w4, x, (((0,), (1,)), ((), ())),
                            preferred_element_type=jnp.float32) + b4_ref[...]
    skipt_ref[...] = qkvst[3 * D:4 * D, :]
    for h in range(H):
        qt = qkvst[h * C:(h + 1) * C, :]
        kt = qkvst[D + h * C:D + (h + 1) * C, :]
        vt = qkvst[2 * D + h * C:2 * D + (h + 1) * C, :]
        qk = lax.dot_general(kt, qt, (((0,), (0,)), ((), ())),
                             preferred_element_type=jnp.float32) * INV
        qkf_ref[h] = qk
        v4_ref[h] = vt
        weh = we_ref[pl.ds(h, 1), :]
        u = lax.dot_general(weh, qt, (((1,), (0,)), ((), ())),
                            preferred_element_type=jnp.float32) * INV
        m_ref[pl.ds(h, 1), :] = jnp.max(qk, axis=0, keepdims=True) + \
            jnp.maximum(u, 0.0)
        u_ref[pl.ds(h, 1), :] = u
        idx_ref[h] = ei_ref[0] * NS + ei_ref[1] + (h * HEADS_BASE)


def _run_k1(x, w4, b4, we2, ei3):
    return pl.pallas_call(
        _k1_body,
        out_shape=[
            jax.ShapeDtypeStruct((H, NS, NS), jnp.float32),
            jax.ShapeDtypeStruct((H, NS), jnp.float32),
            jax.ShapeDtypeStruct((H, NS), jnp.float32),
            jax.ShapeDtypeStruct((H, C, NS), jnp.float32),
            jax.ShapeDtypeStruct((D, NS), jnp.float32),
            jax.ShapeDtypeStruct((H, E // 128, 128), jnp.int32),
        ],
    )(x, w4, b4, we2, ei3)


def _sc_edge(idx3, src2, dst2, a2, qkf_flat, v4_flat, m, u, zeros):
    mesh = plsc.VectorSubcoreMesh(core_axis_name="c", subcore_axis_name="s")

    @functools.partial(
        pl.kernel,
        out_type=[
            jax.ShapeDtypeStruct((32, C * NS), jnp.float32),
            jax.ShapeDtypeStruct((32, NS), jnp.float32),
            jax.ShapeDtypeStruct((32, NS), jnp.float32),
        ],
        mesh=mesh,
        compiler_params=pltpu.CompilerParams(needs_layout_passes=False),
        scratch_types=[
            pltpu.VMEM((ROWS, 128), jnp.int32),
            pltpu.VMEM((ROWS, 128), jnp.int32),
            pltpu.VMEM((ROWS, 128), jnp.int32),
            pltpu.VMEM((ROWS, 128), jnp.float32),
            pltpu.VMEM((ROWS, 128), jnp.float32),
            pltpu.VMEM((C * NS,), jnp.float32),
            pltpu.VMEM((NS,), jnp.float32),
            pltpu.VMEM((NS,), jnp.float32),
            pltpu.VMEM((C * NS,), jnp.float32),
            pltpu.VMEM((NS,), jnp.float32),
            pltpu.VMEM((NS,), jnp.float32),
            pltpu.SemaphoreType.DMA,
        ],
    )
    def k(idx_hbm, src_hbm, dst_hbm, a_hbm, qkf_hbm, v4_hbm, m_hbm, u_hbm,
          z_hbm, nump_hbm, denp_hbm, s1p_hbm,
          idx_v, src_v, dst_v, a_v, qkg_v, vh_v, m_v, u_v, num_v, den_v,
          s1_v, sem):
        wid = lax.axis_index("s") * 2 + lax.axis_index("c")
        h = wid // 8
        slot = wid % 8
        r0 = slot * ROWS
        pltpu.sync_copy(idx_hbm.at[h, pl.ds(r0, ROWS)], idx_v)
        pltpu.sync_copy(src_hbm.at[pl.ds(r0, ROWS)], src_v)
        pltpu.sync_copy(dst_hbm.at[pl.ds(r0, ROWS)], dst_v)
        pltpu.sync_copy(a_hbm.at[pl.ds(r0, ROWS)], a_v)
        pltpu.sync_copy(v4_hbm.at[h], vh_v)
        pltpu.sync_copy(m_hbm.at[h], m_v)
        pltpu.sync_copy(u_hbm.at[h], u_v)
        pltpu.sync_copy(z_hbm, num_v)
        pltpu.sync_copy(z_hbm.at[pl.ds(0, NS)], den_v)
        pltpu.sync_copy(z_hbm.at[pl.ds(0, NS)], s1_v)

        def gbody(i, carry):
            pltpu.async_copy(qkf_hbm.at[idx_v.at[i]], qkg_v.at[i], sem).wait()
            return carry

        lax.fori_loop(0, ROWS, gbody, 0, unroll=False)

        def ebody(i, carry):
            for j in range(8):
                sl = pl.ds(j * 16, 16)
                dstv = dst_v[i, sl]
                av = a_v[i, sl]
                qg = qkg_v[i, sl]
                mv = plsc.load_gather(m_v, [dstv])
                uv = plsc.load_gather(u_v, [dstv])
                ex = jnp.exp(qg - mv + av * uv)
                plsc.addupdate_scatter(den_v, [dstv], ex)
                plsc.addupdate_scatter(s1_v, [dstv], ex * av)
                srcv = src_v[i, sl]
                for c in range(C):
                    vv = plsc.load_gather(vh_v, [srcv + (c * NS)])
                    plsc.addupdate_scatter(num_v, [dstv + (c * NS)], ex * vv)
            return carry

        lax.fori_loop(0, ROWS, ebody, 0, unroll=False)

        pltpu.sync_copy(num_v, nump_hbm.at[wid])
        pltpu.sync_copy(den_v, denp_hbm.at[wid])
        pltpu.sync_copy(s1_v, s1p_hbm.at[wid])

    return k(idx3, src2, dst2, a2, qkf_flat, v4_flat, m, u, zeros)


def _k3_body(nump_ref, denp_ref, s1p_ref, we_ref, skipt_ref, tgtt_ref,
             w42_ref, b42_ref, g1_ref, be1_ref, ms1_ref, g2_ref, be2_ref,
             ms2_ref, out_ref, o1_s, xb_s, o2_s):
    for h in range(H):
        num = jnp.sum(nump_ref[pl.ds(h * 8, 8)], axis=0)
        den = jnp.sum(denp_ref[pl.ds(h * 8, 8)], axis=0, keepdims=True)
        s1 = jnp.sum(s1p_ref[pl.ds(h * 8, 8)], axis=0, keepdims=True)
        wec = we_ref[pl.ds(h * C, C)]
        msg = (num + wec * s1) / (den + 1e-16)
        o1_s[pl.ds(h * C, C), :] = msg
    out1t = o1_s[...] + skipt_ref[...]
    mean1 = jnp.mean(out1t, axis=1, keepdims=True)
    o1 = out1t - ms1_ref[...] * mean1
    var1 = jnp.mean(o1 * o1, axis=1, keepdims=True)
    h1t = jnp.maximum(o1 / jnp.sqrt(var1 + 1e-5) * g1_ref[...] + be1_ref[...],
                      0.0)
    xb_s[:, pl.ds(0, NS)] = h1t
    xb_s[:, pl.ds(NS, NT)] = tgtt_ref[...]
    xbt = xb_s[...]
    qkvs2t = lax.dot_general(w42_ref[...], xbt, (((0,), (0,)), ((), ())),
                             preferred_element_type=jnp.float32) + b42_ref[...]
    skip2t = qkvs2t[3 * D:4 * D, :]
    o2_s[:, pl.ds(0, NS)] = skip2t[:, 0:NS]
    for h in range(H):
        qt = qkvs2t[h * C:(h + 1) * C, NS:NS + NT]
        kt = qkvs2t[D + h * C:D + (h + 1) * C, 0:NS]
        vt = qkvs2t[2 * D + h * C:2 * D + (h + 1) * C, 0:NS]
        s = lax.dot_general(qt, kt, (((0,), (0,)), ((), ())),
                            preferred_element_type=jnp.float32) * INV
        m2 = jnp.max(s, axis=1, keepdims=True)
        aa = jnp.exp(s - m2)
        den2 = jnp.sum(aa, axis=1, keepdims=True)
        aa = aa / (den2 + 1e-16)
        ot = lax.dot_general(vt, aa, (((1,), (1,)), ((), ())),
                             preferred_element_type=jnp.float32)
        o2_s[pl.ds(h * C, C), pl.ds(NS, NT)] = \
            ot + skip2t[h * C:(h + 1) * C, NS:NS + NT]
    out2t = o2_s[...]
    mean2 = jnp.mean(out2t, axis=1, keepdims=True)
    o2 = out2t - ms2_ref[...] * mean2
    var2 = jnp.mean(o2 * o2, axis=1, keepdims=True)
    xt2 = jnp.maximum(o2 / jnp.sqrt(var2 + 1e-5) * g2_ref[...] + be2_ref[...],
                      0.0)
    xtt = xt2[:, NS:NS + NT]
    g = lax.dot_general(xtt, xtt, (((0,), (0,)), ((), ())),
                        preferred_element_type=jnp.float32)
    gmin = jnp.min(g)
    gmax = jnp.max(g)
    out_ref[...] = (g - gmin) / (gmax - gmin + 1e-8)


def _run_k3(nump, denp, s1p, we_col, skipt, tgtt, w42, b42, g1, be1, ms1,
            g2, be2, ms2):
    return pl.pallas_call(
        _k3_body,
        out_shape=jax.ShapeDtypeStruct((NT, NT), jnp.float32),
        scratch_shapes=[
            pltpu.VMEM((D, NS), jnp.float32),
            pltpu.VMEM((D, NS + NT), jnp.float32),
            pltpu.VMEM((D, NS + NT), jnp.float32),
        ],
    )(nump, denp, s1p, we_col, skipt, tgtt, w42, b42, g1, be1, ms1,
      g2, be2, ms2)


def kernel(x, pos_edge_index, edge_attr, target_node_embeddings, params):
    p = params
    w4 = jnp.concatenate([p['Wq1'], p['Wk1'], p['Wv1'], p['Wskip1']], axis=1)
    b4 = jnp.concatenate([p['bq1'], p['bk1'], p['bv1'], p['bskip1']])[:, None]
    we2 = p['We1'].reshape(H, C)
    ei3 = pos_edge_index.reshape(2, E // 128, 128)

    qkf, m, u, v4, skipt, idx = _run_k1(x, w4, b4, we2, ei3)

    src2 = pos_edge_index[0].reshape(E // 128, 128)
    dst2 = pos_edge_index[1].reshape(E // 128, 128)
    a2 = edge_attr.reshape(E // 128, 128)
    zeros = jnp.zeros((C * NS,), jnp.float32)
    nump, denp, s1p = _sc_edge(idx, src2, dst2, a2,
                               qkf.reshape(H * NS * NS),
                               v4.reshape(H, C * NS), m, u, zeros)

    w42 = jnp.concatenate([p['Wq2'], p['Wk2'], p['Wv2'], p['Wskip2']], axis=1)
    b42 = jnp.concatenate([p['bq2'], p['bk2'], p['bv2'], p['bskip2']])[:, None]
    we_col = p['We1'].reshape(D, 1)
    tgtt = target_node_embeddings.T
    nump3 = nump.reshape(32, C, NS)
    return _run_k3(nump3, denp, s1p, we_col, skipt, tgtt, w42, b42,
                   p['g1'][:, None], p['be1'][:, None], p['ms1'][:, None],
                   p['g2'][:, None], p['be2'][:, None], p['ms2'][:, None])


# speedup vs baseline: 308.1867x; 1.1556x over previous
import functools

import jax
import jax.numpy as jnp
from jax import lax
from jax.experimental import pallas as pl
from jax.experimental.pallas import tpu as pltpu
from jax.experimental.pallas import tpu_sc as plsc

H = 4
C = 32
D = H * C
NS = 1024
NT = 256
E = 32768
INV = 1.0 / (C ** 0.5)
HEADS_BASE = NS * NS
EPT = E // 16
ROWS = EPT // 128
ZSL = NS * NS // 16


def _k1_body(x_ref, w4_ref, b4_ref, we_ref, ei_ref,
             qkf_ref, m_ref, u_ref, v4_ref, skipt_ref, idx_ref):
    x = x_ref[...]
    w4 = w4_ref[...]
    qkvst = lax.dot_general(w4, x, (((0,), (1,)), ((), ())),
                            preferred_element_type=jnp.float32) + b4_ref[...]
    skipt_ref[...] = qkvst[3 * D:4 * D, :]
    for h in range(H):
        qt = qkvst[h * C:(h + 1) * C, :]
        kt = qkvst[D + h * C:D + (h + 1) * C, :]
        vt = qkvst[2 * D + h * C:2 * D + (h + 1) * C, :]
        qk = lax.dot_general(kt, qt, (((0,), (0,)), ((), ())),
                             preferred_element_type=jnp.float32) * INV
        qkf_ref[h] = qk
        v4_ref[h] = vt
        weh = we_ref[pl.ds(h, 1), :]
        u = lax.dot_general(weh, qt, (((1,), (0,)), ((), ())),
                            preferred_element_type=jnp.float32) * INV
        m_ref[pl.ds(h, 1), :] = jnp.max(qk, axis=0, keepdims=True) + \
            jnp.maximum(u, 0.0)
        u_ref[pl.ds(h, 1), :] = u
        idx_ref[h] = ei_ref[0] * NS + ei_ref[1] + (h * HEADS_BASE)


def _run_k1(x, w4, b4, we2, ei3):
    return pl.pallas_call(
        _k1_body,
        out_shape=[
            jax.ShapeDtypeStruct((H, NS, NS), jnp.float32),
            jax.ShapeDtypeStruct((H, NS), jnp.float32),
            jax.ShapeDtypeStruct((H, NS), jnp.float32),
            jax.ShapeDtypeStruct((H, C, NS), jnp.float32),
            jax.ShapeDtypeStruct((D, NS), jnp.float32),
            jax.ShapeDtypeStruct((H, E // 128, 128), jnp.int32),
        ],
    )(x, w4, b4, we2, ei3)


def _sc_edge(idx3, a2, qkf_flat, m, u, zeros):
    mesh = plsc.VectorSubcoreMesh(core_axis_name="c", subcore_axis_name="s")

    @functools.partial(
        pl.kernel,
        out_type=[
            jax.ShapeDtypeStruct((2, 2, NS * NS), jnp.float32),
            jax.ShapeDtypeStruct((2, 2, 16, NS), jnp.float32),
        ],
        mesh=mesh,
        compiler_params=pltpu.CompilerParams(needs_layout_passes=False),
        scratch_types=[
            pltpu.VMEM((ROWS, 128), jnp.int32),
            pltpu.VMEM((ROWS, 128), jnp.float32),
            pltpu.VMEM((ROWS, 128), jnp.float32),
            pltpu.VMEM((NS,), jnp.float32),
            pltpu.VMEM((NS,), jnp.float32),
            pltpu.VMEM((EPT,), jnp.float32),
            pltpu.VMEM((EPT,), jnp.int32),
            pltpu.VMEM((NS,), jnp.float32),
            pltpu.VMEM_SHARED((NS * NS,), jnp.float32),
            pltpu.SemaphoreType.DMA,
        ],
    )
    def k(idx_hbm, a_hbm, qkf_hbm, m_hbm, u_hbm, z_hbm,
          p_hbm, s1p_hbm,
          idx_v, a_v, qkg_v, m_v, u_v, p_v, pidx_v, s1_v, P_sh, sem):
        cid = lax.axis_index("c")
        tid = lax.axis_index("s")
        r0 = tid * ROWS
        z16 = jnp.zeros((16,), jnp.float32)
        for kpass in range(2):
            h = 2 * kpass + cid
            hoff = (h * HEADS_BASE).astype(jnp.int32)
            pltpu.sync_copy(z_hbm, P_sh.at[pl.ds(tid * ZSL, ZSL)])
            pltpu.sync_copy(m_hbm.at[h], m_v)
            pltpu.sync_copy(u_hbm.at[h], u_v)
            pltpu.sync_copy(idx_hbm.at[h, pl.ds(r0, ROWS)], idx_v)
            pltpu.sync_copy(a_hbm.at[pl.ds(r0, ROWS)], a_v)
            cps = [pltpu.async_copy(qkf_hbm.at[idx_v.at[i]], qkg_v.at[i], sem)
                   for i in range(ROWS)]
            for cp in cps:
                cp.wait()
            for i in range(NS // 16):
                s1_v[pl.ds(i * 16, 16)] = z16
            for i in range(ROWS):
                for j in range(8):
                    sl = pl.ds(j * 16, 16)
                    qidx = idx_v[i, sl]
                    pidx = qidx - hoff
                    dstv = pidx & (NS - 1)
                    av = a_v[i, sl]
                    qg = qkg_v[i, sl]
                    mv = plsc.load_gather(m_v, [dstv])
                    uv = plsc.load_gather(u_v, [dstv])
                    ex = jnp.exp(qg - mv + av * uv)
                    o = pl.ds((i * 8 + j) * 16, 16)
                    p_v[o] = ex
                    pidx_v[o] = pidx
                    plsc.addupdate_scatter(s1_v, [dstv], ex * av)
            plsc.subcore_barrier()
            pltpu.sync_copy(p_v, P_sh.at[pidx_v], add=True)
            plsc.subcore_barrier()
            pltpu.sync_copy(P_sh.at[pl.ds(tid * ZSL, ZSL)],
                            p_hbm.at[kpass, cid, pl.ds(tid * ZSL, ZSL)])
            pltpu.sync_copy(s1_v, s1p_hbm.at[kpass, cid, tid])

    return k(idx3, a2, qkf_flat, m, u, zeros)


def _k3_body(p_ref, s1p_ref, v4_ref, we_ref, skipt_ref, tgtt_ref,
             w42_ref, b42_ref, g1_ref, be1_ref, ms1_ref, g2_ref, be2_ref,
             ms2_ref, out_ref, o1_s, xb_s, o2_s):
    ones1 = jnp.ones((1, NS), jnp.float32)
    for h in range(H):
        ph = p_ref[h]
        num = lax.dot_general(v4_ref[h], ph, (((1,), (0,)), ((), ())),
                              preferred_element_type=jnp.float32)
        den = lax.dot_general(ones1, ph, (((1,), (0,)), ((), ())),
                              preferred_element_type=jnp.float32)
        s1 = jnp.sum(s1p_ref[h], axis=0, keepdims=True)
        wec = we_ref[pl.ds(h * C, C)]
        msg = (num + wec * s1) / (den + 1e-16)
        o1_s[pl.ds(h * C, C), :] = msg
    out1t = o1_s[...] + skipt_ref[...]
    mean1 = jnp.mean(out1t, axis=1, keepdims=True)
    o1 = out1t - ms1_ref[...] * mean1
    var1 = jnp.mean(o1 * o1, axis=1, keepdims=True)
    h1t = jnp.maximum(o1 / jnp.sqrt(var1 + 1e-5) * g1_ref[...] + be1_ref[...],
                      0.0)
    xb_s[:, pl.ds(0, NS)] = h1t
    xb_s[:, pl.ds(NS, NT)] = tgtt_ref[...]
    xbt = xb_s[...]
    qkvs2t = lax.dot_general(w42_ref[...], xbt, (((0,), (0,)), ((), ())),
                             preferred_element_type=jnp.float32) + b42_ref[...]
    skip2t = qkvs2t[3 * D:4 * D, :]
    o2_s[:, pl.ds(0, NS)] = skip2t[:, 0:NS]
    for h in range(H):
        qt = qkvs2t[h * C:(h + 1) * C, NS:NS + NT]
        kt = qkvs2t[D + h * C:D + (h + 1) * C, 0:NS]
        vt = qkvs2t[2 * D + h * C:2 * D + (h + 1) * C, 0:NS]
        s = lax.dot_general(qt, kt, (((0,), (0,)), ((), ())),
                            preferred_element_type=jnp.float32) * INV
        m2 = jnp.max(s, axis=1, keepdims=True)
        aa = jnp.exp(s - m2)
        den2 = jnp.sum(aa, axis=1, keepdims=True)
        aa = aa / (den2 + 1e-16)
        ot = lax.dot_general(vt, aa, (((1,), (1,)), ((), ())),
                             preferred_element_type=jnp.float32)
        o2_s[pl.ds(h * C, C), pl.ds(NS, NT)] = \
            ot + skip2t[h * C:(h + 1) * C, NS:NS + NT]
    out2t = o2_s[...]
    mean2 = jnp.mean(out2t, axis=1, keepdims=True)
    o2 = out2t - ms2_ref[...] * mean2
    var2 = jnp.mean(o2 * o2, axis=1, keepdims=True)
    xt2 = jnp.maximum(o2 / jnp.sqrt(var2 + 1e-5) * g2_ref[...] + be2_ref[...],
                      0.0)
    xtt = xt2[:, NS:NS + NT]
    g = lax.dot_general(xtt, xtt, (((0,), (0,)), ((), ())),
                        preferred_element_type=jnp.float32)
    gmin = jnp.min(g)
    gmax = jnp.max(g)
    out_ref[...] = (g - gmin) / (gmax - gmin + 1e-8)


def _run_k3(p4, s1p, v4, we_col, skipt, tgtt, w42, b42, g1, be1, ms1,
            g2, be2, ms2):
    return pl.pallas_call(
        _k3_body,
        out_shape=jax.ShapeDtypeStruct((NT, NT), jnp.float32),
        scratch_shapes=[
            pltpu.VMEM((D, NS), jnp.float32),
            pltpu.VMEM((D, NS + NT), jnp.float32),
            pltpu.VMEM((D, NS + NT), jnp.float32),
        ],
    )(p4, s1p, v4, we_col, skipt, tgtt, w42, b42, g1, be1, ms1,
      g2, be2, ms2)


def kernel(x, pos_edge_index, edge_attr, target_node_embeddings, params):
    p = params
    w4 = jnp.concatenate([p['Wq1'], p['Wk1'], p['Wv1'], p['Wskip1']], axis=1)
    b4 = jnp.concatenate([p['bq1'], p['bk1'], p['bv1'], p['bskip1']])[:, None]
    we2 = p['We1'].reshape(H, C)
    ei3 = pos_edge_index.reshape(2, E // 128, 128)

    qkf, m, u, v4, skipt, idx = _run_k1(x, w4, b4, we2, ei3)

    a2 = edge_attr.reshape(E // 128, 128)
    zeros = jnp.zeros((ZSL,), jnp.float32)
    p4, s1p = _sc_edge(idx, a2, qkf.reshape(H * NS * NS), m, u, zeros)

    w42 = jnp.concatenate([p['Wq2'], p['Wk2'], p['Wv2'], p['Wskip2']], axis=1)
    b42 = jnp.concatenate([p['bq2'], p['bk2'], p['bv2'], p['bskip2']])[:, None]
    we_col = p['We1'].reshape(D, 1)
    tgtt = target_node_embeddings.T
    return _run_k3(p4.reshape(H, NS, NS), s1p.reshape(H, 16, NS), v4,
                   we_col, skipt, tgtt, w42, b42,
                   p['g1'][:, None], p['be1'][:, None], p['ms1'][:, None],
                   p['g2'][:, None], p['be2'][:, None], p['ms2'][:, None])


# speedup vs baseline: 323.4755x; 1.0496x over previous
import functools

import jax
import jax.numpy as jnp
from jax import lax
from jax.experimental import pallas as pl
from jax.experimental.pallas import tpu as pltpu
from jax.experimental.pallas import tpu_sc as plsc

H = 4
C = 32
D = H * C
NS = 1024
NT = 256
E = 32768
INV = 1.0 / (C ** 0.5)
HEADS_BASE = NS * NS
EPT = E // 16
ROWS = EPT // 128
ZSL = NS * NS // 16


def _k1_body(x_ref, w4_ref, b4_ref, we_ref, ei_ref,
             qkf_ref, m_ref, u_ref, v4_ref, skipt_ref, idx_ref):
    x = x_ref[...]
    w4 = w4_ref[...]
    qkvst = lax.dot_general(w4, x, (((0,), (1,)), ((), ())),
                            preferred_element_type=jnp.float32) + b4_ref[...]
    skipt_ref[...] = qkvst[3 * D:4 * D, :]
    for h in range(H):
        qt = qkvst[h * C:(h + 1) * C, :]
        kt = qkvst[D + h * C:D + (h + 1) * C, :]
        vt = qkvst[2 * D + h * C:2 * D + (h + 1) * C, :]
        qk = lax.dot_general(kt, qt, (((0,), (0,)), ((), ())),
                             preferred_element_type=jnp.float32) * INV
        for b in range(8):
            qkf_ref[pl.ds(h * NS, NS), b, :] = qk[:, 128 * b:128 * (b + 1)]
        v4_ref[h] = vt
        weh = we_ref[pl.ds(h, 1), :]
        u = lax.dot_general(weh, qt, (((1,), (0,)), ((), ())),
                            preferred_element_type=jnp.float32) * INV
        m_ref[pl.ds(h, 1), :] = jnp.max(qk, axis=0, keepdims=True) + \
            jnp.maximum(u, 0.0)
        u_ref[pl.ds(h, 1), :] = u
        idx_ref[h] = ei_ref[0] * NS + ei_ref[1] + (h * HEADS_BASE)


def _run_k1(x, w4, b4, we2, ei3):
    return pl.pallas_call(
        _k1_body,
        out_shape=[
            jax.ShapeDtypeStruct((H * NS, 8, 128), jnp.float32),
            jax.ShapeDtypeStruct((H, NS), jnp.float32),
            jax.ShapeDtypeStruct((H, NS), jnp.float32),
            jax.ShapeDtypeStruct((H, C, NS), jnp.float32),
            jax.ShapeDtypeStruct((D, NS), jnp.float32),
            jax.ShapeDtypeStruct((H, E // 128, 128), jnp.int32),
        ],
    )(x, w4, b4, we2, ei3)


def _sc_edge(idx3, a2, qkf_flat, m, u, zeros):
    mesh = plsc.VectorSubcoreMesh(core_axis_name="c", subcore_axis_name="s")

    @functools.partial(
        pl.kernel,
        out_type=[
            jax.ShapeDtypeStruct((2, 2, NS * NS), jnp.float32),
            jax.ShapeDtypeStruct((2, 2, 16, NS), jnp.float32),
        ],
        mesh=mesh,
        compiler_params=pltpu.CompilerParams(needs_layout_passes=False),
        scratch_types=[
            pltpu.VMEM((ROWS, 128), jnp.int32),
            pltpu.VMEM((ROWS, 128), jnp.float32),
            pltpu.VMEM((ROWS, 128), jnp.float32),
            pltpu.VMEM((NS,), jnp.float32),
            pltpu.VMEM((NS,), jnp.float32),
            pltpu.VMEM((EPT,), jnp.float32),
            pltpu.VMEM((EPT,), jnp.int32),
            pltpu.VMEM((NS,), jnp.float32),
            pltpu.VMEM_SHARED((NS * NS,), jnp.float32),
            pltpu.SemaphoreType.DMA,
        ],
    )
    def k(idx_hbm, a_hbm, qkf_hbm, m_hbm, u_hbm, z_hbm,
          p_hbm, s1p_hbm,
          idx_v, a_v, qkg_v, m_v, u_v, p_v, pidx_v, s1_v, P_sh, sem):
        cid = lax.axis_index("c")
        tid = lax.axis_index("s")
        r0 = tid * ROWS
        z16 = jnp.zeros((16,), jnp.float32)
        for kpass in range(2):
            h = 2 * kpass + cid
            hoff = (h * HEADS_BASE).astype(jnp.int32)
            pltpu.sync_copy(z_hbm, P_sh.at[pl.ds(tid * ZSL, ZSL)])
            pltpu.sync_copy(m_hbm.at[h], m_v)
            pltpu.sync_copy(u_hbm.at[h], u_v)
            pltpu.sync_copy(idx_hbm.at[h, pl.ds(r0, ROWS)], idx_v)
            pltpu.sync_copy(a_hbm.at[pl.ds(r0, ROWS)], a_v)
            cps = [pltpu.async_copy(qkf_hbm.at[idx_v.at[i]], qkg_v.at[i], sem)
                   for i in range(ROWS)]
            for cp in cps:
                cp.wait()
            for i in range(NS // 16):
                s1_v[pl.ds(i * 16, 16)] = z16
            for i in range(ROWS):
                for j in range(8):
                    sl = pl.ds(j * 16, 16)
                    qidx = idx_v[i, sl]
                    pidx = qidx - hoff
                    dstv = pidx & (NS - 1)
                    av = a_v[i, sl]
                    qg = qkg_v[i, sl]
                    mv = plsc.load_gather(m_v, [dstv])
                    uv = plsc.load_gather(u_v, [dstv])
                    ex = jnp.exp(qg - mv + av * uv)
                    o = pl.ds((i * 8 + j) * 16, 16)
                    p_v[o] = ex
                    pidx_v[o] = pidx
                    plsc.addupdate_scatter(s1_v, [dstv], ex * av)
            plsc.subcore_barrier()
            pltpu.sync_copy(p_v, P_sh.at[pidx_v], add=True)
            plsc.subcore_barrier()
            pltpu.sync_copy(P_sh.at[pl.ds(tid * ZSL, ZSL)],
                            p_hbm.at[kpass, cid, pl.ds(tid * ZSL, ZSL)])
            pltpu.sync_copy(s1_v, s1p_hbm.at[kpass, cid, tid])

    return k(idx3, a2, qkf_flat, m, u, zeros)


def _k3_body(p_ref, s1p_ref, v4_ref, we_ref, skipt_ref, tgtt_ref,
             w42_ref, b42_ref, g1_ref, be1_ref, ms1_ref, g2_ref, be2_ref,
             ms2_ref, out_ref, o1_s, xb_s, o2_s):
    ones1 = jnp.ones((1, NS), jnp.float32)
    for h in range(H):
        vh = v4_ref[h]
        s1 = jnp.sum(s1p_ref[h], axis=0, keepdims=True)
        wec = we_ref[pl.ds(h * C, C)]
        for b in range(8):
            phb = p_ref[pl.ds(h * NS, NS), b, :]
            num = lax.dot_general(vh, phb, (((1,), (0,)), ((), ())),
                                  preferred_element_type=jnp.float32)
            den = lax.dot_general(ones1, phb, (((1,), (0,)), ((), ())),
                                  preferred_element_type=jnp.float32)
            s1b = s1[:, 128 * b:128 * (b + 1)]
            msg = (num + wec * s1b) / (den + 1e-16)
            o1_s[pl.ds(h * C, C), pl.ds(128 * b, 128)] = msg
    out1t = o1_s[...] + skipt_ref[...]
    mean1 = jnp.mean(out1t, axis=1, keepdims=True)
    o1 = out1t - ms1_ref[...] * mean1
    var1 = jnp.mean(o1 * o1, axis=1, keepdims=True)
    h1t = jnp.maximum(o1 / jnp.sqrt(var1 + 1e-5) * g1_ref[...] + be1_ref[...],
                      0.0)
    xb_s[:, pl.ds(0, NS)] = h1t
    xb_s[:, pl.ds(NS, NT)] = tgtt_ref[...]
    xbt = xb_s[...]
    qkvs2t = lax.dot_general(w42_ref[...], xbt, (((0,), (0,)), ((), ())),
                             preferred_element_type=jnp.float32) + b42_ref[...]
    skip2t = qkvs2t[3 * D:4 * D, :]
    o2_s[:, pl.ds(0, NS)] = skip2t[:, 0:NS]
    for h in range(H):
        qt = qkvs2t[h * C:(h + 1) * C, NS:NS + NT]
        kt = qkvs2t[D + h * C:D + (h + 1) * C, 0:NS]
        vt = qkvs2t[2 * D + h * C:2 * D + (h + 1) * C, 0:NS]
        s = lax.dot_general(qt, kt, (((0,), (0,)), ((), ())),
                            preferred_element_type=jnp.float32) * INV
        m2 = jnp.max(s, axis=1, keepdims=True)
        aa = jnp.exp(s - m2)
        den2 = jnp.sum(aa, axis=1, keepdims=True)
        aa = aa / (den2 + 1e-16)
        ot = lax.dot_general(vt, aa, (((1,), (1,)), ((), ())),
                             preferred_element_type=jnp.float32)
        o2_s[pl.ds(h * C, C), pl.ds(NS, NT)] = \
            ot + skip2t[h * C:(h + 1) * C, NS:NS + NT]
    out2t = o2_s[...]
    mean2 = jnp.mean(out2t, axis=1, keepdims=True)
    o2 = out2t - ms2_ref[...] * mean2
    var2 = jnp.mean(o2 * o2, axis=1, keepdims=True)
    xt2 = jnp.maximum(o2 / jnp.sqrt(var2 + 1e-5) * g2_ref[...] + be2_ref[...],
                      0.0)
    xtt = xt2[:, NS:NS + NT]
    g = lax.dot_general(xtt, xtt, (((0,), (0,)), ((), ())),
                        preferred_element_type=jnp.float32)
    gmin = jnp.min(g)
    gmax = jnp.max(g)
    out_ref[...] = (g - gmin) / (gmax - gmin + 1e-8)


def _run_k3(p4, s1p, v4, we_col, skipt, tgtt, w42, b42, g1, be1, ms1,
            g2, be2, ms2):
    return pl.pallas_call(
        _k3_body,
        out_shape=jax.ShapeDtypeStruct((NT, NT), jnp.float32),
        scratch_shapes=[
            pltpu.VMEM((D, NS), jnp.float32),
            pltpu.VMEM((D, NS + NT), jnp.float32),
            pltpu.VMEM((D, NS + NT), jnp.float32),
        ],
    )(p4, s1p, v4, we_col, skipt, tgtt, w42, b42, g1, be1, ms1,
      g2, be2, ms2)


def kernel(x, pos_edge_index, edge_attr, target_node_embeddings, params):
    p = params
    w4 = jnp.concatenate([p['Wq1'], p['Wk1'], p['Wv1'], p['Wskip1']], axis=1)
    b4 = jnp.concatenate([p['bq1'], p['bk1'], p['bv1'], p['bskip1']])[:, None]
    we2 = p['We1'].reshape(H, C)
    ei3 = pos_edge_index.reshape(2, E // 128, 128)

    qkf, m, u, v4, skipt, idx = _run_k1(x, w4, b4, we2, ei3)

    a2 = edge_attr.reshape(E // 128, 128)
    zeros = jnp.zeros((ZSL,), jnp.float32)
    p4, s1p = _sc_edge(idx, a2, qkf.reshape(H * NS * NS), m, u, zeros)

    w42 = jnp.concatenate([p['Wq2'], p['Wk2'], p['Wv2'], p['Wskip2']], axis=1)
    b42 = jnp.concatenate([p['bq2'], p['bk2'], p['bv2'], p['bskip2']])[:, None]
    we_col = p['We1'].reshape(D, 1)
    tgtt = target_node_embeddings.T
    return _run_k3(p4.reshape(H * NS, 8, 128), s1p.reshape(H, 16, NS), v4,
                   we_col, skipt, tgtt, w42, b42,
                   p['g1'][:, None], p['be1'][:, None], p['ms1'][:, None],
                   p['g2'][:, None], p['be2'][:, None], p['ms2'][:, None])


# speedup vs baseline: 345.8673x; 1.0692x over previous
import functools

import jax
import jax.numpy as jnp
from jax import lax
from jax.experimental import pallas as pl
from jax.experimental.pallas import tpu as pltpu
from jax.experimental.pallas import tpu_sc as plsc

H = 4
C = 32
D = H * C
NS = 1024
NT = 256
E = 32768
INV = 1.0 / (C ** 0.5)
HEADS_BASE = NS * NS
EPT = E // 16
ROWS = EPT // 128
ZSL = NS * NS // 16


def _k1_body(x_ref, w4_ref, b4_ref, we_ref, ei_ref,
             qkf_ref, m_ref, u_ref, v4_ref, skipt_ref, idx_ref):
    x = x_ref[...]
    w4 = w4_ref[...]
    qkvst = lax.dot_general(w4, x, (((0,), (1,)), ((), ())),
                            preferred_element_type=jnp.float32) + b4_ref[...]
    skipt_ref[...] = qkvst[3 * D:4 * D, :]
    for h in range(H):
        qt = qkvst[h * C:(h + 1) * C, :]
        kt = qkvst[D + h * C:D + (h + 1) * C, :]
        vt = qkvst[2 * D + h * C:2 * D + (h + 1) * C, :]
        qk = lax.dot_general(kt, qt, (((0,), (0,)), ((), ())),
                             preferred_element_type=jnp.float32) * INV
        for b in range(8):
            qkf_ref[pl.ds(h * NS, NS), b, :] = qk[:, 128 * b:128 * (b + 1)]
        v4_ref[h] = vt
        weh = we_ref[pl.ds(h, 1), :]
        u = lax.dot_general(weh, qt, (((1,), (0,)), ((), ())),
                            preferred_element_type=jnp.float32) * INV
        m_ref[pl.ds(h, 1), :] = jnp.max(qk, axis=0, keepdims=True) + \
            jnp.maximum(u, 0.0)
        u_ref[pl.ds(h, 1), :] = u
        idx_ref[h] = ei_ref[0] * NS + ei_ref[1] + (h * HEADS_BASE)


def _run_k1(x, w4, b4, we2, ei3):
    return pl.pallas_call(
        _k1_body,
        out_shape=[
            jax.ShapeDtypeStruct((H * NS, 8, 128), jnp.float32),
            jax.ShapeDtypeStruct((H, NS), jnp.float32),
            jax.ShapeDtypeStruct((H, NS), jnp.float32),
            jax.ShapeDtypeStruct((H, C, NS), jnp.float32),
            jax.ShapeDtypeStruct((D, NS), jnp.float32),
            jax.ShapeDtypeStruct((H, E // 128, 128), jnp.int32),
        ],
    )(x, w4, b4, we2, ei3)


def _sc_edge(idx3, a2, qkf_flat, m, u, zeros):
    mesh = plsc.VectorSubcoreMesh(core_axis_name="c", subcore_axis_name="s")

    @functools.partial(
        pl.kernel,
        out_type=[
            jax.ShapeDtypeStruct((2, 2, NS * NS), jnp.float32),
            jax.ShapeDtypeStruct((2, 2, 16, NS), jnp.float32),
        ],
        mesh=mesh,
        compiler_params=pltpu.CompilerParams(needs_layout_passes=False),
        scratch_types=[
            pltpu.VMEM((ROWS, 128), jnp.int32),
            pltpu.VMEM((ROWS, 128), jnp.float32),
            pltpu.VMEM((ROWS, 128), jnp.float32),
            pltpu.VMEM((NS,), jnp.float32),
            pltpu.VMEM((NS,), jnp.float32),
            pltpu.VMEM((EPT,), jnp.float32),
            pltpu.VMEM((EPT,), jnp.float32),
            pltpu.VMEM((EPT,), jnp.int32),
            pltpu.VMEM((EPT,), jnp.int32),
            pltpu.VMEM((NS,), jnp.float32),
            pltpu.VMEM_SHARED((NS * NS,), jnp.float32),
            pltpu.SemaphoreType.DMA,
        ],
    )
    def k(idx_hbm, a_hbm, qkf_hbm, m_hbm, u_hbm, z_hbm,
          p_hbm, s1p_hbm,
          idx_v, a_v, qkg_v, m_v, u_v, p0_v, p1_v, pidx0_v, pidx1_v,
          s1_v, P_sh, sem):
        cid = lax.axis_index("c")
        tid = lax.axis_index("s")
        r0 = tid * ROWS
        z16 = jnp.zeros((16,), jnp.float32)
        myslc = pl.ds(tid * ZSL, ZSL)

        def compute_pass(kpass, p_v, pidx_v):
            h = 2 * kpass + cid
            hoff = (h * HEADS_BASE).astype(jnp.int32)
            pltpu.sync_copy(m_hbm.at[h], m_v)
            pltpu.sync_copy(u_hbm.at[h], u_v)
            pltpu.sync_copy(idx_hbm.at[h, pl.ds(r0, ROWS)], idx_v)
            cps = [pltpu.async_copy(qkf_hbm.at[idx_v.at[i]], qkg_v.at[i], sem)
                   for i in range(ROWS)]
            for cp in cps:
                cp.wait()
            for i in range(NS // 16):
                s1_v[pl.ds(i * 16, 16)] = z16
            for i in range(ROWS):
                for j in range(8):
                    sl = pl.ds(j * 16, 16)
                    qidx = idx_v[i, sl]
                    pidx = qidx - hoff
                    dstv = pidx & (NS - 1)
                    av = a_v[i, sl]
                    qg = qkg_v[i, sl]
                    mv = plsc.load_gather(m_v, [dstv])
                    uv = plsc.load_gather(u_v, [dstv])
                    ex = jnp.exp(qg - mv + av * uv)
                    o = pl.ds((i * 8 + j) * 16, 16)
                    p_v[o] = ex
                    pidx_v[o] = pidx
                    plsc.addupdate_scatter(s1_v, [dstv], ex * av)
            pltpu.sync_copy(s1_v, s1p_hbm.at[kpass, cid, tid])

        zcp = pltpu.async_copy(z_hbm, P_sh.at[myslc], sem)
        pltpu.sync_copy(a_hbm.at[pl.ds(r0, ROWS)], a_v)
        compute_pass(0, p0_v, pidx0_v)
        zcp.wait()
        plsc.subcore_barrier()
        pltpu.sync_copy(p0_v, P_sh.at[pidx0_v], add=True)
        plsc.subcore_barrier()
        dcp = pltpu.async_copy(P_sh.at[myslc], p_hbm.at[0, cid, myslc], sem)
        compute_pass(1, p1_v, pidx1_v)
        dcp.wait()
        pltpu.sync_copy(z_hbm, P_sh.at[myslc])
        plsc.subcore_barrier()
        pltpu.sync_copy(p1_v, P_sh.at[pidx1_v], add=True)
        plsc.subcore_barrier()
        pltpu.sync_copy(P_sh.at[myslc], p_hbm.at[1, cid, myslc])

    return k(idx3, a2, qkf_flat, m, u, zeros)


def _k3_body(p_ref, s1p_ref, v4_ref, we_ref, skipt_ref, tgtt_ref,
             w42_ref, b42_ref, g1_ref, be1_ref, ms1_ref, g2_ref, be2_ref,
             ms2_ref, out_ref, o1_s, xb_s, o2_s):
    ones1 = jnp.ones((1, NS), jnp.float32)
    for h in range(H):
        vh = v4_ref[h]
        s1 = jnp.sum(s1p_ref[h], axis=0, keepdims=True)
        wec = we_ref[pl.ds(h * C, C)]
        for b in range(8):
            phb = p_ref[pl.ds(h * NS, NS), b, :]
            num = lax.dot_general(vh, phb, (((1,), (0,)), ((), ())),
                                  preferred_element_type=jnp.float32)
            den = lax.dot_general(ones1, phb, (((1,), (0,)), ((), ())),
                                  preferred_element_type=jnp.float32)
            s1b = s1[:, 128 * b:128 * (b + 1)]
            msg = (num + wec * s1b) / (den + 1e-16)
            o1_s[pl.ds(h * C, C), pl.ds(128 * b, 128)] = msg
    out1t = o1_s[...] + skipt_ref[...]
    mean1 = jnp.mean(out1t, axis=1, keepdims=True)
    o1 = out1t - ms1_ref[...] * mean1
    var1 = jnp.mean(o1 * o1, axis=1, keepdims=True)
    h1t = jnp.maximum(o1 / jnp.sqrt(var1 + 1e-5) * g1_ref[...] + be1_ref[...],
                      0.0)
    xb_s[:, pl.ds(0, NS)] = h1t
    xb_s[:, pl.ds(NS, NT)] = tgtt_ref[...]
    xbt = xb_s[...]
    qkvs2t = lax.dot_general(w42_ref[...], xbt, (((0,), (0,)), ((), ())),
                             preferred_element_type=jnp.float32) + b42_ref[...]
    skip2t = qkvs2t[3 * D:4 * D, :]
    o2_s[:, pl.ds(0, NS)] = skip2t[:, 0:NS]
    for h in range(H):
        qt = qkvs2t[h * C:(h + 1) * C, NS:NS + NT]
        kt = qkvs2t[D + h * C:D + (h + 1) * C, 0:NS]
        vt = qkvs2t[2 * D + h * C:2 * D + (h + 1) * C, 0:NS]
        s = lax.dot_general(qt, kt, (((0,), (0,)), ((), ())),
                            preferred_element_type=jnp.float32) * INV
        m2 = jnp.max(s, axis=1, keepdims=True)
        aa = jnp.exp(s - m2)
        den2 = jnp.sum(aa, axis=1, keepdims=True)
        aa = aa / (den2 + 1e-16)
        ot = lax.dot_general(vt, aa, (((1,), (1,)), ((), ())),
                             preferred_element_type=jnp.float32)
        o2_s[pl.ds(h * C, C), pl.ds(NS, NT)] = \
            ot + skip2t[h * C:(h + 1) * C, NS:NS + NT]
    out2t = o2_s[...]
    mean2 = jnp.mean(out2t, axis=1, keepdims=True)
    o2 = out2t - ms2_ref[...] * mean2
    var2 = jnp.mean(o2 * o2, axis=1, keepdims=True)
    xt2 = jnp.maximum(o2 / jnp.sqrt(var2 + 1e-5) * g2_ref[...] + be2_ref[...],
                      0.0)
    xtt = xt2[:, NS:NS + NT]
    g = lax.dot_general(xtt, xtt, (((0,), (0,)), ((), ())),
                        preferred_element_type=jnp.float32)
    gmin = jnp.min(g)
    gmax = jnp.max(g)
    out_ref[...] = (g - gmin) / (gmax - gmin + 1e-8)


def _run_k3(p4, s1p, v4, we_col, skipt, tgtt, w42, b42, g1, be1, ms1,
            g2, be2, ms2):
    return pl.pallas_call(
        _k3_body,
        out_shape=jax.ShapeDtypeStruct((NT, NT), jnp.float32),
        scratch_shapes=[
            pltpu.VMEM((D, NS), jnp.float32),
            pltpu.VMEM((D, NS + NT), jnp.float32),
            pltpu.VMEM((D, NS + NT), jnp.float32),
        ],
    )(p4, s1p, v4, we_col, skipt, tgtt, w42, b42, g1, be1, ms1,
      g2, be2, ms2)


def kernel(x, pos_edge_index, edge_attr, target_node_embeddings, params):
    p = params
    w4 = jnp.concatenate([p['Wq1'], p['Wk1'], p['Wv1'], p['Wskip1']], axis=1)
    b4 = jnp.concatenate([p['bq1'], p['bk1'], p['bv1'], p['bskip1']])[:, None]
    we2 = p['We1'].reshape(H, C)
    ei3 = pos_edge_index.reshape(2, E // 128, 128)

    qkf, m, u, v4, skipt, idx = _run_k1(x, w4, b4, we2, ei3)

    a2 = edge_attr.reshape(E // 128, 128)
    zeros = jnp.zeros((ZSL,), jnp.float32)
    p4, s1p = _sc_edge(idx, a2, qkf.reshape(H * NS * NS), m, u, zeros)

    w42 = jnp.concatenate([p['Wq2'], p['Wk2'], p['Wv2'], p['Wskip2']], axis=1)
    b42 = jnp.concatenate([p['bq2'], p['bk2'], p['bv2'], p['bskip2']])[:, None]
    we_col = p['We1'].reshape(D, 1)
    tgtt = target_node_embeddings.T
    return _run_k3(p4.reshape(H * NS, 8, 128), s1p.reshape(H, 16, NS), v4,
                   we_col, skipt, tgtt, w42, b42,
                   p['g1'][:, None], p['be1'][:, None], p['ms1'][:, None],
                   p['g2'][:, None], p['be2'][:, None], p['ms2'][:, None])


# speedup vs baseline: 348.1133x; 1.0065x over previous
import functools

import jax
import jax.numpy as jnp
from jax import lax
from jax.experimental import pallas as pl
from jax.experimental.pallas import tpu as pltpu
from jax.experimental.pallas import tpu_sc as plsc

H = 4
C = 32
D = H * C
NS = 1024
NT = 256
E = 32768
INV = 1.0 / (C ** 0.5)
HEADS_BASE = NS * NS
EPT = E // 16
ROWS = EPT // 128
ZSL = NS * NS // 16


def _k1a_body(x_ref, w2_ref, b2_ref, we_ref, ei_ref,
              qkf_ref, m_ref, u_ref, idx_ref):
    x = x_ref[...]
    w2 = w2_ref[...]
    qkt = lax.dot_general(w2, x, (((0,), (1,)), ((), ())),
                          preferred_element_type=jnp.float32) + b2_ref[...]
    for h in range(H):
        qt = qkt[h * C:(h + 1) * C, :]
        kt = qkt[D + h * C:D + (h + 1) * C, :]
        qk = lax.dot_general(kt, qt, (((0,), (0,)), ((), ())),
                             preferred_element_type=jnp.float32) * INV
        for b in range(8):
            qkf_ref[pl.ds(h * NS, NS), b, :] = qk[:, 128 * b:128 * (b + 1)]
        weh = we_ref[pl.ds(h, 1), :]
        u = lax.dot_general(weh, qt, (((1,), (0,)), ((), ())),
                            preferred_element_type=jnp.float32) * INV
        m_ref[pl.ds(h, 1), :] = jnp.max(qk, axis=0, keepdims=True) + \
            jnp.maximum(u, 0.0)
        u_ref[pl.ds(h, 1), :] = u
        idx_ref[h] = ei_ref[0] * NS + ei_ref[1] + (h * HEADS_BASE)


def _run_k1a(x, w2, b2, we2, ei3):
    return pl.pallas_call(
        _k1a_body,
        out_shape=[
            jax.ShapeDtypeStruct((H * NS, 8, 128), jnp.float32),
            jax.ShapeDtypeStruct((H, NS), jnp.float32),
            jax.ShapeDtypeStruct((H, NS), jnp.float32),
            jax.ShapeDtypeStruct((H, E // 128, 128), jnp.int32),
        ],
    )(x, w2, b2, we2, ei3)


def _k1b_body(x_ref, w2_ref, b2_ref, v4_ref, skipt_ref):
    x = x_ref[...]
    w2 = w2_ref[...]
    vst = lax.dot_general(w2, x, (((0,), (1,)), ((), ())),
                          preferred_element_type=jnp.float32) + b2_ref[...]
    for h in range(H):
        v4_ref[h] = vst[h * C:(h + 1) * C, :]
    skipt_ref[...] = vst[D:2 * D, :]


def _run_k1b(x, w2, b2):
    return pl.pallas_call(
        _k1b_body,
        out_shape=[
            jax.ShapeDtypeStruct((H, C, NS), jnp.float32),
            jax.ShapeDtypeStruct((D, NS), jnp.float32),
        ],
    )(x, w2, b2)


def _sc_edge(idx3, a2, qkf_flat, m, u, zeros):
    mesh = plsc.VectorSubcoreMesh(core_axis_name="c", subcore_axis_name="s")

    @functools.partial(
        pl.kernel,
        out_type=[
            jax.ShapeDtypeStruct((2, 2, NS * NS), jnp.float32),
            jax.ShapeDtypeStruct((2, 2, 16, NS), jnp.float32),
        ],
        mesh=mesh,
        compiler_params=pltpu.CompilerParams(needs_layout_passes=False),
        scratch_types=[
            pltpu.VMEM((ROWS, 128), jnp.int32),
            pltpu.VMEM((ROWS, 128), jnp.float32),
            pltpu.VMEM((ROWS, 128), jnp.float32),
            pltpu.VMEM((NS,), jnp.float32),
            pltpu.VMEM((NS,), jnp.float32),
            pltpu.VMEM((EPT,), jnp.float32),
            pltpu.VMEM((EPT,), jnp.float32),
            pltpu.VMEM((EPT,), jnp.int32),
            pltpu.VMEM((EPT,), jnp.int32),
            pltpu.VMEM((NS,), jnp.float32),
            pltpu.VMEM_SHARED((NS * NS,), jnp.float32),
            pltpu.SemaphoreType.DMA,
        ],
    )
    def k(idx_hbm, a_hbm, qkf_hbm, m_hbm, u_hbm, z_hbm,
          p_hbm, s1p_hbm,
          idx_v, a_v, qkg_v, m_v, u_v, p0_v, p1_v, pidx0_v, pidx1_v,
          s1_v, P_sh, sem):
        cid = lax.axis_index("c")
        tid = lax.axis_index("s")
        r0 = tid * ROWS
        z16 = jnp.zeros((16,), jnp.float32)
        myslc = pl.ds(tid * ZSL, ZSL)

        def compute_pass(kpass, p_v, pidx_v):
            h = 2 * kpass + cid
            hoff = (h * HEADS_BASE).astype(jnp.int32)
            pltpu.sync_copy(m_hbm.at[h], m_v)
            pltpu.sync_copy(u_hbm.at[h], u_v)
            pltpu.sync_copy(idx_hbm.at[h, pl.ds(r0, ROWS)], idx_v)
            cps = [pltpu.async_copy(qkf_hbm.at[idx_v.at[i]], qkg_v.at[i], sem)
                   for i in range(ROWS)]
            for cp in cps:
                cp.wait()
            for i in range(NS // 16):
                s1_v[pl.ds(i * 16, 16)] = z16
            for i in range(ROWS):
                for j in range(8):
                    sl = pl.ds(j * 16, 16)
                    qidx = idx_v[i, sl]
                    pidx = qidx - hoff
                    dstv = pidx & (NS - 1)
                    av = a_v[i, sl]
                    qg = qkg_v[i, sl]
                    mv = plsc.load_gather(m_v, [dstv])
                    uv = plsc.load_gather(u_v, [dstv])
                    ex = jnp.exp(qg - mv + av * uv)
                    o = pl.ds((i * 8 + j) * 16, 16)
                    p_v[o] = ex
                    pidx_v[o] = pidx
                    plsc.addupdate_scatter(s1_v, [dstv], ex * av)
            pltpu.sync_copy(s1_v, s1p_hbm.at[kpass, cid, tid])

        zcp = pltpu.async_copy(z_hbm, P_sh.at[myslc], sem)
        pltpu.sync_copy(a_hbm.at[pl.ds(r0, ROWS)], a_v)
        compute_pass(0, p0_v, pidx0_v)
        zcp.wait()
        plsc.subcore_barrier()
        pltpu.sync_copy(p0_v, P_sh.at[pidx0_v], add=True)
        plsc.subcore_barrier()
        dcp = pltpu.async_copy(P_sh.at[myslc], p_hbm.at[0, cid, myslc], sem)
        compute_pass(1, p1_v, pidx1_v)
        dcp.wait()
        pltpu.sync_copy(z_hbm, P_sh.at[myslc])
        plsc.subcore_barrier()
        pltpu.sync_copy(p1_v, P_sh.at[pidx1_v], add=True)
        plsc.subcore_barrier()
        pltpu.sync_copy(P_sh.at[myslc], p_hbm.at[1, cid, myslc])

    return k(idx3, a2, qkf_flat, m, u, zeros)


def _k3_body(p_ref, s1p_ref, v4_ref, we_ref, skipt_ref, tgtt_ref,
             w42_ref, b42_ref, g1_ref, be1_ref, ms1_ref, g2_ref, be2_ref,
             ms2_ref, out_ref, o1_s, xb_s, o2_s):
    ones1 = jnp.ones((1, NS), jnp.float32)
    for h in range(H):
        vh = v4_ref[h]
        s1 = jnp.sum(s1p_ref[h], axis=0, keepdims=True)
        wec = we_ref[pl.ds(h * C, C)]
        for b in range(8):
            phb = p_ref[pl.ds(h * NS, NS), b, :]
            num = lax.dot_general(vh, phb, (((1,), (0,)), ((), ())),
                                  preferred_element_type=jnp.float32)
            den = lax.dot_general(ones1, phb, (((1,), (0,)), ((), ())),
                                  preferred_element_type=jnp.float32)
            s1b = s1[:, 128 * b:128 * (b + 1)]
            msg = (num + wec * s1b) / (den + 1e-16)
            o1_s[pl.ds(h * C, C), pl.ds(128 * b, 128)] = msg
    out1t = o1_s[...] + skipt_ref[...]
    mean1 = jnp.mean(out1t, axis=1, keepdims=True)
    o1 = out1t - ms1_ref[...] * mean1
    var1 = jnp.mean(o1 * o1, axis=1, keepdims=True)
    h1t = jnp.maximum(o1 / jnp.sqrt(var1 + 1e-5) * g1_ref[...] + be1_ref[...],
                      0.0)
    xb_s[:, pl.ds(0, NS)] = h1t
    xb_s[:, pl.ds(NS, NT)] = tgtt_ref[...]
    xbt = xb_s[...]
    qkvs2t = lax.dot_general(w42_ref[...], xbt, (((0,), (0,)), ((), ())),
                             preferred_element_type=jnp.float32) + b42_ref[...]
    skip2t = qkvs2t[3 * D:4 * D, :]
    o2_s[:, pl.ds(0, NS)] = skip2t[:, 0:NS]
    for h in range(H):
        qt = qkvs2t[h * C:(h + 1) * C, NS:NS + NT]
        kt = qkvs2t[D + h * C:D + (h + 1) * C, 0:NS]
        vt = qkvs2t[2 * D + h * C:2 * D + (h + 1) * C, 0:NS]
        s = lax.dot_general(qt, kt, (((0,), (0,)), ((), ())),
                            preferred_element_type=jnp.float32) * INV
        m2 = jnp.max(s, axis=1, keepdims=True)
        aa = jnp.exp(s - m2)
        den2 = jnp.sum(aa, axis=1, keepdims=True)
        aa = aa / (den2 + 1e-16)
        ot = lax.dot_general(vt, aa, (((1,), (1,)), ((), ())),
                             preferred_element_type=jnp.float32)
        o2_s[pl.ds(h * C, C), pl.ds(NS, NT)] = \
            ot + skip2t[h * C:(h + 1) * C, NS:NS + NT]
    out2t = o2_s[...]
    mean2 = jnp.mean(out2t, axis=1, keepdims=True)
    o2 = out2t - ms2_ref[...] * mean2
    var2 = jnp.mean(o2 * o2, axis=1, keepdims=True)
    xt2 = jnp.maximum(o2 / jnp.sqrt(var2 + 1e-5) * g2_ref[...] + be2_ref[...],
                      0.0)
    xtt = xt2[:, NS:NS + NT]
    g = lax.dot_general(xtt, xtt, (((0,), (0,)), ((), ())),
                        preferred_element_type=jnp.float32)
    gmin = jnp.min(g)
    gmax = jnp.max(g)
    out_ref[...] = (g - gmin) / (gmax - gmin + 1e-8)


def _run_k3(p4, s1p, v4, we_col, skipt, tgtt, w42, b42, g1, be1, ms1,
            g2, be2, ms2):
    return pl.pallas_call(
        _k3_body,
        out_shape=jax.ShapeDtypeStruct((NT, NT), jnp.float32),
        scratch_shapes=[
            pltpu.VMEM((D, NS), jnp.float32),
            pltpu.VMEM((D, NS + NT), jnp.float32),
            pltpu.VMEM((D, NS + NT), jnp.float32),
        ],
    )(p4, s1p, v4, we_col, skipt, tgtt, w42, b42, g1, be1, ms1,
      g2, be2, ms2)


def kernel(x, pos_edge_index, edge_attr, target_node_embeddings, params):
    p = params
    wqk = jnp.concatenate([p['Wq1'], p['Wk1']], axis=1)
    bqk = jnp.concatenate([p['bq1'], p['bk1']])[:, None]
    wvs = jnp.concatenate([p['Wv1'], p['Wskip1']], axis=1)
    bvs = jnp.concatenate([p['bv1'], p['bskip1']])[:, None]
    we2 = p['We1'].reshape(H, C)
    ei3 = pos_edge_index.reshape(2, E // 128, 128)

    qkf, m, u, idx = _run_k1a(x, wqk, bqk, we2, ei3)

    a2 = edge_attr.reshape(E // 128, 128)
    zeros = jnp.zeros((ZSL,), jnp.float32)
    p4, s1p = _sc_edge(idx, a2, qkf.reshape(H * NS * NS), m, u, zeros)
    v4, skipt = _run_k1b(x, wvs, bvs)

    w42 = jnp.concatenate([p['Wq2'], p['Wk2'], p['Wv2'], p['Wskip2']], axis=1)
    b42 = jnp.concatenate([p['bq2'], p['bk2'], p['bv2'], p['bskip2']])[:, None]
    we_col = p['We1'].reshape(D, 1)
    tgtt = target_node_embeddings.T
    return _run_k3(p4.reshape(H * NS, 8, 128), s1p.reshape(H, 16, NS), v4,
                   we_col, skipt, tgtt, w42, b42,
                   p['g1'][:, None], p['be1'][:, None], p['ms1'][:, None],
                   p['g2'][:, None], p['be2'][:, None], p['ms2'][:, None])
